# Initial kernel scaffold; baseline (speedup 1.0000x reference)
import jax
import jax.numpy as jnp
from jax.experimental import pallas as pl


def kernel(xyz_template, xyz_partial, params):
    raise NotImplementedError("write your pallas kernel here")


# speedup vs baseline: 10.8576x; 10.8576x over previous
import functools
import math

import jax
import jax.numpy as jnp
import numpy as np
from jax import lax
from jax.experimental import pallas as pl
from jax.experimental.pallas import tpu as pltpu
from jax.experimental.pallas import tpu_sc as plsc

_BN_SCALE = 1.0 / math.sqrt(1.0 + 1e-5)
_SQ = np.sqrt(np.float32(1.0 + 1e-5))

_SC_NC = 2
_SC_NS = 16
_SC_NW = _SC_NC * _SC_NS


def _prep(p, bn=True):
    arrs = [p['W'].T, p['b'][None, :]]
    if bn:
        arrs += [p['g'][None, :], p['be'][None, :]]
    return arrs


def _apply_layers(h, refs, bn_flags, relu_flags):
    o = 0
    for bn, relu in zip(bn_flags, relu_flags):
        w = refs[o][...]
        b = refs[o + 1][...]
        o += 2
        h = jnp.dot(h, w, preferred_element_type=jnp.float32) + b
        if bn:
            g = refs[o][...]
            be = refs[o + 1][...]
            o += 2
            h = (h / _SQ) * g + be
        if relu:
            h = jnp.maximum(h, 0.0)
    return h


def _pad_last(x, d):
    c = x.shape[-1]
    if c == d:
        return x
    pad = [(0, 0)] * (x.ndim - 1) + [(0, d - c)]
    return jnp.pad(x, pad)


def _fps_body(xyz_ref, out_ref, *, npoint):
    B, _, N = xyz_ref.shape
    x = xyz_ref[:, 0, :]
    y = xyz_ref[:, 1, :]
    z = xyz_ref[:, 2, :]
    iota = lax.broadcasted_iota(jnp.int32, (B, N), 1)
    iota_np = lax.broadcasted_iota(jnp.int32, (B, npoint), 1)

    zero_np = (iota_np * 0)

    def body(i, state):
        dist, far = state
        out_ref[...] = jnp.where(iota_np == i, far + zero_np, out_ref[...])
        sel = (iota == far).astype(jnp.float32)
        cx = jnp.sum(x * sel, axis=1, keepdims=True)
        cy = jnp.sum(y * sel, axis=1, keepdims=True)
        cz = jnp.sum(z * sel, axis=1, keepdims=True)
        d = (x - cx) ** 2 + (y - cy) ** 2 + (z - cz) ** 2
        dist = jnp.minimum(dist, d)
        m = jnp.max(dist, axis=1, keepdims=True)
        far = jnp.min(jnp.where(dist == m, iota, N), axis=1, keepdims=True)
        return dist, far

    out_ref[...] = zero_np
    dist0 = x * 0.0 + 1e10
    far0 = (iota[:, :1] * 0)
    lax.fori_loop(0, npoint, body, (dist0, far0))


def _fps(xyz_t, npoint):
    B = xyz_t.shape[0]
    return pl.pallas_call(
        functools.partial(_fps_body, npoint=npoint),
        out_shape=jax.ShapeDtypeStruct((B, npoint), jnp.int32),
    )(xyz_t)


def _sqdist(c, P):
    return (jnp.sum(c * c, axis=1, keepdims=True)
            + jnp.sum(P * P, axis=0, keepdims=True)
            - 2.0 * jnp.dot(c, P, preferred_element_type=jnp.float32))


def _ballq_body(nxyz_ref, xyzt_ref, out_ref, *, nsample, radius2):
    Sb = nxyz_ref.shape[1]
    N = xyzt_ref.shape[2]
    c = nxyz_ref[0]
    P = xyzt_ref[0]
    d = _sqdist(c, P)
    big = float(N)
    iota = lax.broadcasted_iota(jnp.int32, (Sb, N), 1).astype(jnp.float32)
    cand = jnp.where(d <= radius2, iota, big)
    iota_k = lax.broadcasted_iota(jnp.int32, (Sb, nsample), 1)
    g = jnp.full((Sb, nsample), big, jnp.float32)
    first = None
    for k in range(nsample):
        m = jnp.min(cand, axis=1, keepdims=True)
        if first is None:
            first = m
        g = jnp.where(iota_k == k, m + jnp.zeros_like(g), g)
        cand = jnp.where(cand == m, big, cand)
    g = jnp.where(g == big, first + jnp.zeros_like(g), g)
    out_ref[0] = g.astype(jnp.int32)


def _ball_query(new_xyz, xyz_t, radius, nsample, s_blk):
    B, S, _ = new_xyz.shape
    N = xyz_t.shape[2]
    grid = (B, S // s_blk)
    return pl.pallas_call(
        functools.partial(_ballq_body, nsample=nsample, radius2=radius * radius),
        grid=grid,
        in_specs=[
            pl.BlockSpec((1, s_blk, 3), lambda b, s: (b, s, 0)),
            pl.BlockSpec((1, 3, N), lambda b, s: (b, 0, 0)),
        ],
        out_specs=pl.BlockSpec((1, s_blk, nsample), lambda b, s: (b, s, 0)),
        out_shape=jax.ShapeDtypeStruct((B, S, nsample), jnp.int32),
    )(new_xyz, xyz_t)


def _sc_gather(table, idx):
    M = idx.shape[0]
    D = table.shape[1]
    assert D % 16 == 0 and M % (8 * _SC_NW) == 0
    b_per_w = M // _SC_NW
    cr = min(b_per_w, 128)
    nch = b_per_w // cr
    row_bytes = 4 * D
    gb = 1
    for cand_gb in (4, 2):
        if nch % cand_gb == 0 and cand_gb * cr * row_bytes <= 450_000:
            gb = cand_gb
            break
    ngroups = nch // gb
    idx3 = idx.reshape(_SC_NW, nch, cr)
    mesh = plsc.VectorSubcoreMesh(core_axis_name="c", subcore_axis_name="s")

    @functools.partial(
        pl.kernel, mesh=mesh,
        compiler_params=pltpu.CompilerParams(use_tc_tiling_on_sc=False),
        out_type=jax.ShapeDtypeStruct((M, D), jnp.float32),
        scratch_types=[
            pltpu.VMEM((nch, cr), jnp.int32),
            pltpu.VMEM((gb * cr, D), jnp.float32),
            pltpu.SemaphoreType.DMA,
        ],
    )
    def k(table_hbm, idx_hbm, out_hbm, idx_v, rows_v, sem):
        wid = lax.axis_index("s") * _SC_NC + lax.axis_index("c")
        base = wid * b_per_w
        pltpu.sync_copy(idx_hbm.at[wid], idx_v)

        def group(g, carry):
            g0 = g * gb
            copies = []
            for k_ in range(gb):
                copies.append(pltpu.async_copy(
                    table_hbm.at[idx_v.at[g0 + k_]],
                    rows_v.at[pl.ds(k_ * cr, cr)], sem))
            for c in copies:
                c.wait()
            pltpu.sync_copy(rows_v, out_hbm.at[pl.ds(base + g0 * cr, gb * cr)])
            return carry

        lax.fori_loop(0, ngroups, group, 0)

    return k(table, idx3)


def _gather_rows(table_flat, idx, out_rows_shape):
    g = _sc_gather(table_flat, idx.reshape(-1))
    return g.reshape(out_rows_shape + (table_flat.shape[1],))


def _sa_mlp_body(g_ref, c_ref, *refs, K, nlayers):
    w_refs = refs[:-1]
    out_ref = refs[-1]
    M = g_ref.shape[1]
    Sb = M // K
    X = g_ref[0]
    cexp = c_ref[0]
    Dp = X.shape[1]
    E = (lax.broadcasted_iota(jnp.int32, (3, Dp), 0)
         == lax.broadcasted_iota(jnp.int32, (3, Dp), 1)).astype(jnp.float32)
    cpad = jnp.dot(cexp, E, precision=jax.lax.Precision.HIGHEST,
                   preferred_element_type=jnp.float32)
    h = _apply_layers(X - cpad, w_refs, (True,) * nlayers, (True,) * nlayers)
    out_ref[0] = jnp.max(h.reshape(Sb, K, h.shape[1]), axis=1)


def _sa_mlp(g3, cexp, K, layer_arrs, s_blk):
    B, M, Dp = g3.shape
    S = M // K
    flat = [a for la in layer_arrs for a in la]
    C3 = layer_arrs[-1][0].shape[1]
    return pl.pallas_call(
        functools.partial(_sa_mlp_body, K=K, nlayers=len(layer_arrs)),
        grid=(B, S // s_blk),
        in_specs=[
            pl.BlockSpec((1, s_blk * K, Dp), lambda b, s: (b, s, 0)),
            pl.BlockSpec((1, s_blk * K, 3), lambda b, s: (b, s, 0)),
        ] + [pl.BlockSpec(a.shape, lambda b, s: (0, 0)) for a in flat],
        out_specs=pl.BlockSpec((1, s_blk, C3), lambda b, s: (b, s, 0)),
        out_shape=jax.ShapeDtypeStruct((B, S, C3), jnp.float32),
    )(g3, cexp, *flat)


def _fp_body(x1_ref, x2t_ref, p2_ref, p1_ref, *refs, bn_flags, relu_flags):
    w_refs = refs[:-1]
    out_ref = refs[-1]
    Nb = x1_ref.shape[1]
    N2 = x2t_ref.shape[2]
    c = x1_ref[0]
    P = x2t_ref[0]
    d = _sqdist(c, P)
    iota = lax.broadcasted_iota(jnp.int32, (Nb, N2), 1).astype(jnp.float32)
    cand = d
    idxs, vals = [], []
    for _ in range(3):
        m = jnp.min(cand, axis=1, keepdims=True)
        am = jnp.min(jnp.where(cand == m, iota, float(N2)), axis=1, keepdims=True)
        idxs.append(am)
        vals.append(m)
        cand = jnp.where(iota == am, jnp.inf, cand)
    r0 = 1.0 / (vals[0] + 1e-8)
    r1 = 1.0 / (vals[1] + 1e-8)
    r2 = 1.0 / (vals[2] + 1e-8)
    norm = r0 + r1 + r2
    p2 = p2_ref[0]
    hp = jax.lax.Precision.HIGHEST
    f0 = jnp.dot((iota == idxs[0]).astype(jnp.float32), p2,
                 precision=hp, preferred_element_type=jnp.float32)
    f1 = jnp.dot((iota == idxs[1]).astype(jnp.float32), p2,
                 precision=hp, preferred_element_type=jnp.float32)
    f2 = jnp.dot((iota == idxs[2]).astype(jnp.float32), p2,
                 precision=hp, preferred_element_type=jnp.float32)
    interp = f0 * (r0 / norm) + f1 * (r1 / norm) + f2 * (r2 / norm)
    h = jnp.concatenate([p1_ref[0], interp], axis=1)
    out_ref[0] = _apply_layers(h, w_refs, bn_flags, relu_flags)


def _fp(xyz1, xyz2_t, pts2, pts1, layer_arrs, bn_flags, relu_flags, n_blk):
    B, N1, _ = xyz1.shape
    N2 = xyz2_t.shape[2]
    C2 = pts2.shape[2]
    C1 = pts1.shape[2]
    Cout = layer_arrs[-1][0].shape[1]
    flat = [a for la in layer_arrs for a in la]
    specs = [
        pl.BlockSpec((1, n_blk, 3), lambda b, n: (b, n, 0)),
        pl.BlockSpec((1, 3, N2), lambda b, n: (b, 0, 0)),
        pl.BlockSpec((1, N2, C2), lambda b, n: (b, 0, 0)),
        pl.BlockSpec((1, n_blk, C1), lambda b, n: (b, n, 0)),
    ] + [pl.BlockSpec(a.shape, lambda b, n: (0, 0)) for a in flat]
    return pl.pallas_call(
        functools.partial(_fp_body, bn_flags=bn_flags, relu_flags=relu_flags),
        grid=(B, N1 // n_blk),
        in_specs=specs,
        out_specs=pl.BlockSpec((1, n_blk, Cout), lambda b, n: (b, n, 0)),
        out_shape=jax.ShapeDtypeStruct((B, N1, Cout), jnp.float32),
    )(xyz1, xyz2_t, pts2, pts1, *flat)


def _stack_body(x_ref, *refs, nlayers, relu_flags, use_trans):
    if use_trans:
        t_ref = refs[0]
        refs = refs[1:]
    w_refs = refs[:-1]
    out_ref = refs[-1]
    h = x_ref[0]
    if use_trans:
        h = jnp.dot(h, t_ref[0], preferred_element_type=jnp.float32)
    h = _apply_layers(h, w_refs, (True,) * nlayers, relu_flags)
    out_ref[0, 0] = jnp.max(h, axis=0)


def _stack_maxpool(x, layer_arrs, relu_flags, trans=None):
    B, Np, _ = x.shape
    Cout = layer_arrs[-1][0].shape[1]
    use_trans = trans is not None
    args = [x]
    specs = [pl.BlockSpec((1, Np, x.shape[2]), lambda b: (b, 0, 0))]
    if use_trans:
        args.append(trans)
        specs.append(pl.BlockSpec((1, 3, 3), lambda b: (b, 0, 0)))
    flat = [a for la in layer_arrs for a in la]
    args += flat
    specs += [pl.BlockSpec(a.shape, lambda b: (0, 0)) for a in flat]
    return pl.pallas_call(
        functools.partial(_stack_body, nlayers=len(layer_arrs),
                          relu_flags=relu_flags, use_trans=use_trans),
        grid=(B,),
        in_specs=specs,
        out_specs=pl.BlockSpec((1, 1, Cout), lambda b: (b, 0, 0)),
        out_shape=jax.ShapeDtypeStruct((B, 1, Cout), jnp.float32),
    )(*args)[:, 0, :]


def _stn_fc_body(h_ref, *refs):
    w_refs = refs[:-1]
    out_ref = refs[-1]
    t = _apply_layers(h_ref[...], w_refs, (True, True, False),
                      (True, True, False))
    iden = (lax.broadcasted_iota(jnp.int32, t.shape, 1) % 4 == 0).astype(jnp.float32)
    out_ref[...] = t + iden


def _stn_fc(h, layer_arrs):
    flat = [a for la in layer_arrs for a in la]
    B = h.shape[0]
    return pl.pallas_call(
        _stn_fc_body,
        out_shape=jax.ShapeDtypeStruct((B, 9), jnp.float32),
    )(h, *flat)


def _unc_body(p_ref, l3_ref, *refs):
    w_refs = refs[:-1]
    out_ref = refs[-1]
    m = jnp.max(l3_ref[...], axis=1)
    x = jnp.concatenate([p_ref[...], m], axis=1)
    out_ref[...] = _apply_layers(x, w_refs, (True, False), (True, False))


def _unc_head(partial_feats, l3_points, layer_arrs):
    flat = [a for la in layer_arrs for a in la]
    B = partial_feats.shape[0]
    return pl.pallas_call(
        _unc_body,
        out_shape=jax.ShapeDtypeStruct((B, 2), jnp.float32),
    )(partial_feats, l3_points, *flat)


def _sa_level(xyz, pts, layers_p, npoint, radius, nsample, s_blk_q, s_blk_m):
    B, N, _ = xyz.shape
    C = pts.shape[2]
    Dp = ((3 + C) + 15) // 16 * 16
    xyz_t = jnp.transpose(xyz, (0, 2, 1))
    table = _pad_last(jnp.concatenate([xyz, pts], axis=-1), Dp)
    table_flat = table.reshape(B * N, Dp)
    boff = (jnp.arange(B, dtype=jnp.int32) * N)[:, None]

    fps_idx = _fps(xyz_t, npoint)
    new_rows = _gather_rows(table_flat, fps_idx + boff, (B, npoint))
    new_xyz = new_rows[:, :, :3]
    idx = _ball_query(new_xyz, xyz_t, radius, nsample, s_blk_q)
    idx = jnp.minimum(idx, N - 1)
    g = _sc_gather(table_flat, (idx + boff[:, :, None]).reshape(-1))
    g3 = g.reshape(B, npoint * nsample, Dp)
    cexp = jnp.broadcast_to(new_xyz[:, :, None, :], (B, npoint, nsample, 3))
    cexp = cexp.reshape(B, npoint * nsample, 3)
    layer_arrs = [_prep(p) for p in layers_p]
    w0 = layer_arrs[0][0]
    layer_arrs[0][0] = jnp.pad(w0, ((0, Dp - w0.shape[0]), (0, 0)))
    new_pts = _sa_mlp(g3, cexp, nsample, layer_arrs, s_blk_m)
    return new_xyz, new_pts


def kernel(xyz_template, xyz_partial, params):
    B = xyz_template.shape[0]
    l0_xyz = jnp.transpose(xyz_template, (0, 2, 1))
    l0_pts = l0_xyz

    l1_xyz, l1_pts = _sa_level(l0_xyz, l0_pts, params['sa1'], 512, 0.2, 32,
                               s_blk_q=256, s_blk_m=128)
    l2_xyz, l2_pts = _sa_level(l1_xyz, l1_pts, params['sa2'], 128, 0.4, 64,
                               s_blk_q=128, s_blk_m=32)
    l3_xyz, l3_pts = _sa_level(l2_xyz, l2_pts, params['sa3'], 32, 0.8, 128,
                               s_blk_q=32, s_blk_m=8)

    xp = jnp.transpose(xyz_partial, (0, 2, 1))
    stn = params['enc']['stn']
    h = _stack_maxpool(xp, [_prep(stn['c1']), _prep(stn['c2']), _prep(stn['c3'])],
                       [True, True, True])
    trans = _stn_fc(h, [_prep(stn['f1']), _prep(stn['f2']), _prep(stn['f3'], bn=False)])
    trans = trans.reshape(B, 3, 3)
    enc = params['enc']
    partial_feats = _stack_maxpool(
        xp, [_prep(enc['c1']), _prep(enc['c2']), _prep(enc['c3'])],
        [True, True, False], trans=trans)

    rep = jnp.broadcast_to(partial_feats[:, None, :], (B, 128, 1024))
    pts1_fp3 = jnp.concatenate([rep, l2_xyz, l2_pts], axis=-1)
    fp3_layers = [_prep(p) for p in params['fp3']]
    l2_new = _fp(l2_xyz, jnp.transpose(l3_xyz, (0, 2, 1)), l3_pts, pts1_fp3,
                 fp3_layers, (True, True), (True, True), n_blk=128)
    fp2_layers = [_prep(p) for p in params['fp2']]
    l1_new = _fp(l1_xyz, jnp.transpose(l2_xyz, (0, 2, 1)), l2_new, l1_pts,
                 fp2_layers, (True, True), (True, True), n_blk=512)
    fp1_layers = ([_prep(p) for p in params['fp1']]
                  + [_prep(params['head1']), _prep(params['head2'], bn=False)])
    flow = _fp(l0_xyz, jnp.transpose(l1_xyz, (0, 2, 1)), l1_new, l0_pts,
               fp1_layers, (True, True, True, True, False),
               (True, True, True, True, False), n_blk=512)

    unc = _unc_head(partial_feats, l3_pts,
                    [_prep(params['unc1']), _prep(params['unc2'], bn=False)])
    return jnp.transpose(flow, (0, 2, 1)), unc[:, :, None]
